# Initial kernel scaffold; baseline (speedup 1.0000x reference)
import jax
import jax.numpy as jnp
from jax.experimental import pallas as pl


def kernel(x, pos, neighbor_idx, W, b):
    raise NotImplementedError("write your pallas kernel here")


# speedup vs baseline: 44.2062x; 44.2062x over previous
import functools

import jax
import jax.numpy as jnp
from jax import lax
from jax.experimental import pallas as pl
from jax.experimental.pallas import tpu as pltpu
from jax.experimental.pallas import tpu_sc as plsc

B, N, K, C, O = 8, 4096, 32, 64, 64
BN = B * N
LANES = 16
NW = 32
PPW = BN // NW
WPB = N // PPW
TPC = 4
CHUNK = TPC * K
NCHUNK = PPW // TPC


def _mm_body(x_ref, pos_ref, wf_ref, wp_ref, b_ref, z_ref, c_ref):
    cpart = jnp.dot(pos_ref[...], wp_ref[...],
                    preferred_element_type=jnp.float32)
    z_ref[...] = (jnp.dot(x_ref[...], wf_ref[...],
                          preferred_element_type=jnp.float32)
                  + cpart + b_ref[...])
    c_ref[...] = cpart


def _tc_mm(x2, pos2, wf, wp, bias):
    BLK = 2048
    return pl.pallas_call(
        _mm_body,
        grid=(BN // BLK,),
        in_specs=[
            pl.BlockSpec((BLK, C), lambda i: (i, 0)),
            pl.BlockSpec((BLK, 8), lambda i: (i, 0)),
            pl.BlockSpec((C, O), lambda i: (0, 0)),
            pl.BlockSpec((8, O), lambda i: (0, 0)),
            pl.BlockSpec((1, O), lambda i: (0, 0)),
        ],
        out_specs=[
            pl.BlockSpec((BLK, O), lambda i: (i, 0)),
            pl.BlockSpec((BLK, O), lambda i: (i, 0)),
        ],
        out_shape=[
            jax.ShapeDtypeStruct((BN, O), jnp.float32),
            jax.ShapeDtypeStruct((BN, O), jnp.float32),
        ],
    )(x2, pos2, wf, wp, bias)


def _sc_body(z_hbm, c_hbm, idx_hbm, out_hbm, idx_v, rows_v, c_v, out_v, sem):
    nc = 2
    wid = lax.axis_index("s") * nc + lax.axis_index("c")
    bb = wid // WPB
    lp_base = (wid % WPB) * PPW
    gidx_base = wid * (PPW * K)

    def chunk(g, carry):
        lp0 = lp_base + g * TPC
        pltpu.sync_copy(idx_hbm.at[pl.ds(gidx_base + g * CHUNK, CHUNK)],
                        idx_v)
        pltpu.async_copy(z_hbm.at[bb].at[idx_v], rows_v, sem).wait()
        pltpu.sync_copy(c_hbm.at[bb].at[pl.ds(lp0, TPC)], c_v)
        for t in range(TPC):
            for j in range(O // LANES):
                sl = pl.ds(j * LANES, LANES)
                acc = rows_v[t * K, sl]
                for k in range(1, K):
                    acc = jnp.maximum(acc, rows_v[t * K + k, sl])
                out_v[t, sl] = jnp.maximum(acc - c_v[t, sl], 0.0)
        pltpu.sync_copy(out_v, out_hbm.at[bb].at[pl.ds(lp0, TPC)])
        return carry

    lax.fori_loop(0, NCHUNK, chunk, 0)


_sc_gathermax = functools.partial(
    pl.kernel,
    out_type=jax.ShapeDtypeStruct((B, N, O), jnp.float32),
    mesh=plsc.VectorSubcoreMesh(core_axis_name="c", subcore_axis_name="s"),
    scratch_types=[
        pltpu.VMEM((CHUNK,), jnp.int32),
        pltpu.VMEM((CHUNK, O), jnp.float32),
        pltpu.VMEM((TPC, O), jnp.float32),
        pltpu.VMEM((TPC, O), jnp.float32),
        pltpu.SemaphoreType.DMA,
    ],
    compiler_params=pltpu.CompilerParams(use_tc_tiling_on_sc=False),
)(_sc_body)


def kernel(x, pos, neighbor_idx, W, b):
    x2 = x.reshape(BN, C)
    pos2 = jnp.pad(pos.reshape(BN, 3), ((0, 0), (0, 5)))
    wf = W[3:]
    wp = jnp.pad(W[:3], ((0, 5), (0, 0)))
    z2, c2 = _tc_mm(x2, pos2, wf, wp, b.reshape(1, O))
    z = z2.reshape(B, N, O)
    c = c2.reshape(B, N, O)
    idx_flat = neighbor_idx.reshape(BN * K)
    out = _sc_gathermax(z, c, idx_flat)
    return out


# speedup vs baseline: 113.8965x; 2.5765x over previous
import functools

import jax
import jax.numpy as jnp
from jax import lax
from jax.experimental import pallas as pl
from jax.experimental.pallas import tpu as pltpu
from jax.experimental.pallas import tpu_sc as plsc

B, N, K, C, O = 8, 4096, 32, 64, 64
BN = B * N
LANES = 16
NW = 32
PPW = BN // NW
WPB = N // PPW
TPC = 4
CHUNK = TPC * K
NCHUNK = PPW // TPC


def _mm_body(x_ref, pos_ref, wf_ref, wp_ref, b_ref, z_ref, c_ref):
    cpart = jnp.dot(pos_ref[...], wp_ref[...],
                    preferred_element_type=jnp.float32)
    z_ref[...] = (jnp.dot(x_ref[...], wf_ref[...],
                          preferred_element_type=jnp.float32)
                  + cpart + b_ref[...])
    c_ref[...] = cpart


def _tc_mm(x2, pos2, wf, wp, bias):
    BLK = 2048
    return pl.pallas_call(
        _mm_body,
        grid=(BN // BLK,),
        in_specs=[
            pl.BlockSpec((BLK, C), lambda i: (i, 0)),
            pl.BlockSpec((BLK, 8), lambda i: (i, 0)),
            pl.BlockSpec((C, O), lambda i: (0, 0)),
            pl.BlockSpec((8, O), lambda i: (0, 0)),
            pl.BlockSpec((1, O), lambda i: (0, 0)),
        ],
        out_specs=[
            pl.BlockSpec((BLK, O), lambda i: (i, 0)),
            pl.BlockSpec((BLK, O), lambda i: (i, 0)),
        ],
        out_shape=[
            jax.ShapeDtypeStruct((BN, O), jnp.float32),
            jax.ShapeDtypeStruct((BN, O), jnp.float32),
        ],
    )(x2, pos2, wf, wp, bias)


SUP = 128
CPS = SUP // TPC
NSUP = PPW // SUP
NPAIR = NCHUNK // 2


def _sc_body(z_hbm, c_hbm, idx_hbm, out_hbm, idx_all, rows_v, c_v, out_v,
             semg0, semg1):
    nc = 2
    wid = lax.axis_index("s") * nc + lax.axis_index("c")
    bb = wid // WPB
    lp_base = (wid % WPB) * PPW
    ztab = z_hbm.at[bb]
    sems = (semg0, semg1)

    pltpu.sync_copy(idx_hbm.at[wid], idx_all)

    def fire(ci, b):
        pltpu.async_copy(ztab.at[idx_all.at[ci]], rows_v.at[b], sems[b])

    def drain(b):
        pltpu.make_async_copy(ztab.at[pl.ds(0, CHUNK)], rows_v.at[b],
                              sems[b]).wait()

    def compute(ci, b):
        base = (ci % CPS) * TPC
        for t in range(TPC):
            for j in range(O // LANES):
                sl = pl.ds(j * LANES, LANES)
                acc = rows_v[b, t * K, sl]
                for k in range(1, K):
                    acc = jnp.maximum(acc, rows_v[b, t * K + k, sl])
                out_v[base + t, sl] = jnp.maximum(acc - c_v[base + t, sl],
                                                  0.0)

    fire(0, 0)

    def pair(p, carry):
        ci0 = 2 * p
        sup = p // (CPS // 2)

        @pl.when(p % (CPS // 2) == 0)
        def _():
            pltpu.sync_copy(c_hbm.at[bb].at[pl.ds(lp_base + sup * SUP, SUP)],
                            c_v)

        fire(ci0 + 1, 1)
        drain(0)
        compute(ci0, 0)

        @pl.when(ci0 + 2 < NCHUNK)
        def _():
            fire(ci0 + 2, 0)

        drain(1)
        compute(ci0 + 1, 1)

        @pl.when(p % (CPS // 2) == (CPS // 2) - 1)
        def _():
            pltpu.sync_copy(out_v,
                            out_hbm.at[bb].at[pl.ds(lp_base + sup * SUP,
                                                    SUP)])
        return carry

    lax.fori_loop(0, NPAIR, pair, 0)


_sc_gathermax = functools.partial(
    pl.kernel,
    out_type=jax.ShapeDtypeStruct((B, N, O), jnp.float32),
    mesh=plsc.VectorSubcoreMesh(core_axis_name="c", subcore_axis_name="s"),
    scratch_types=[
        pltpu.VMEM((NCHUNK, CHUNK), jnp.int32),
        pltpu.VMEM((2, CHUNK, O), jnp.float32),
        pltpu.VMEM((SUP, O), jnp.float32),
        pltpu.VMEM((SUP, O), jnp.float32),
        pltpu.SemaphoreType.DMA,
        pltpu.SemaphoreType.DMA,
    ],
    compiler_params=pltpu.CompilerParams(use_tc_tiling_on_sc=False),
)(_sc_body)


def kernel(x, pos, neighbor_idx, W, b):
    x2 = x.reshape(BN, C)
    pos2 = jnp.pad(pos.reshape(BN, 3), ((0, 0), (0, 5)))
    wf = W[3:]
    wp = jnp.pad(W[:3], ((0, 5), (0, 0)))
    z2, c2 = _tc_mm(x2, pos2, wf, wp, b.reshape(1, O))
    z = z2.reshape(B, N, O)
    c = c2.reshape(B, N, O)
    idx_blk = neighbor_idx.reshape(NW, NCHUNK, CHUNK)
    out = _sc_gathermax(z, c, idx_blk)
    return out


# speedup vs baseline: 126.1654x; 1.1077x over previous
import functools

import jax
import jax.numpy as jnp
import numpy as np
from jax import lax
from jax.experimental import pallas as pl
from jax.experimental.pallas import tpu as pltpu
from jax.experimental.pallas import tpu_sc as plsc

B, N, K, C, O = 8, 4096, 32, 64, 64
BN = B * N
LANES = 16
NW = 32
PPW = BN // NW
WPB = N // PPW
TPC = 4
CHUNK = TPC * K
NCHUNK = PPW // TPC


def _mm_body(x_ref, pos_ref, wfp_ref, wpp_ref, wp_ref, b_ref,
             z_ref, c_ref):
    zp = (jnp.dot(x_ref[...], wfp_ref[...],
                  preferred_element_type=jnp.float32)
          + jnp.dot(pos_ref[...], wpp_ref[...],
                    preferred_element_type=jnp.float32)
          + b_ref[...])
    z_ref[...] = zp.astype(jnp.bfloat16)
    c_ref[...] = jnp.dot(pos_ref[...], wp_ref[...],
                         preferred_element_type=jnp.float32)


def _tc_mm(x2, pos2, wfp, wpp, wp, bias):
    BLK = 2048
    return pl.pallas_call(
        _mm_body,
        grid=(BN // BLK,),
        in_specs=[
            pl.BlockSpec((BLK, C), lambda i: (i, 0)),
            pl.BlockSpec((BLK, 8), lambda i: (i, 0)),
            pl.BlockSpec((C, O), lambda i: (0, 0)),
            pl.BlockSpec((8, O), lambda i: (0, 0)),
            pl.BlockSpec((8, O), lambda i: (0, 0)),
            pl.BlockSpec((1, O), lambda i: (0, 0)),
        ],
        out_specs=[
            pl.BlockSpec((BLK, O), lambda i: (i, 0)),
            pl.BlockSpec((BLK, O), lambda i: (i, 0)),
        ],
        out_shape=[
            jax.ShapeDtypeStruct((BN, O), jnp.bfloat16),
            jax.ShapeDtypeStruct((BN, O), jnp.float32),
        ],
    )(x2, pos2, wfp, wpp, wp, bias)


SUP = 128
CPS = SUP // TPC
NSUP = PPW // SUP
NPAIR = NCHUNK // 2


def _sc_body(z_hbm, c_hbm, idx_hbm, out_hbm, idx_all, rows_v, c_v, out_v,
             semg0, semg1):
    nc = 2
    wid = lax.axis_index("s") * nc + lax.axis_index("c")
    bb = wid // WPB
    lp_base = (wid % WPB) * PPW
    ztab = z_hbm.at[bb]
    sems = (semg0, semg1)

    pltpu.sync_copy(idx_hbm.at[wid], idx_all)

    def fire(ci, b):
        pltpu.async_copy(ztab.at[idx_all.at[ci]], rows_v.at[b], sems[b])

    def drain(b):
        pltpu.make_async_copy(ztab.at[pl.ds(0, CHUNK)], rows_v.at[b],
                              sems[b]).wait()

    def compute(ci, b):
        base = (ci % CPS) * TPC
        for t in range(TPC):
            for g in range(O // 32):
                sl = pl.ds(g * 32, 32)
                acc = rows_v[b, t * K, sl]
                for k in range(1, K):
                    acc = jnp.maximum(acc, rows_v[b, t * K + k, sl])
                lo, hi = plsc.unpack(acc, format=plsc.PackFormat.INTERLEAVED)
                sl_lo = pl.ds(g * 32, LANES)
                sl_hi = pl.ds(g * 32 + LANES, LANES)
                out_v[base + t, sl_lo] = jnp.maximum(
                    lo - c_v[base + t, sl_lo], 0.0)
                out_v[base + t, sl_hi] = jnp.maximum(
                    hi - c_v[base + t, sl_hi], 0.0)

    fire(0, 0)

    def pair(p, carry):
        ci0 = 2 * p
        sup = p // (CPS // 2)

        @pl.when(p % (CPS // 2) == 0)
        def _():
            pltpu.sync_copy(c_hbm.at[bb].at[pl.ds(lp_base + sup * SUP, SUP)],
                            c_v)

        fire(ci0 + 1, 1)
        drain(0)
        compute(ci0, 0)

        @pl.when(ci0 + 2 < NCHUNK)
        def _():
            fire(ci0 + 2, 0)

        drain(1)
        compute(ci0 + 1, 1)

        @pl.when(p % (CPS // 2) == (CPS // 2) - 1)
        def _():
            pltpu.sync_copy(out_v,
                            out_hbm.at[bb].at[pl.ds(lp_base + sup * SUP,
                                                    SUP)])
        return carry

    lax.fori_loop(0, NPAIR, pair, 0)


_sc_gathermax = functools.partial(
    pl.kernel,
    out_type=jax.ShapeDtypeStruct((B, N, O), jnp.float32),
    mesh=plsc.VectorSubcoreMesh(core_axis_name="c", subcore_axis_name="s"),
    scratch_types=[
        pltpu.VMEM((NCHUNK, CHUNK), jnp.int32),
        pltpu.VMEM((2, CHUNK, O), jnp.bfloat16),
        pltpu.VMEM((SUP, O), jnp.float32),
        pltpu.VMEM((SUP, O), jnp.float32),
        pltpu.SemaphoreType.DMA,
        pltpu.SemaphoreType.DMA,
    ],
    compiler_params=pltpu.CompilerParams(use_tc_tiling_on_sc=False,
                                         needs_layout_passes=False),
)(_sc_body)


_PERM = np.empty(O, dtype=np.int32)
for _g in range(O // 32):
    _PERM[_g * 32 + 0:_g * 32 + 32:2] = np.arange(16) + _g * 32
    _PERM[_g * 32 + 1:_g * 32 + 32:2] = np.arange(16) + _g * 32 + 16


def kernel(x, pos, neighbor_idx, W, b):
    x2 = x.reshape(BN, C)
    pos2 = jnp.pad(pos.reshape(BN, 3), ((0, 0), (0, 5)))
    wf = W[3:]
    wp = jnp.pad(W[:3], ((0, 5), (0, 0)))
    wfp = wf[:, _PERM]
    wpp = wp[:, _PERM]
    z2, c2 = _tc_mm(x2, pos2, wfp, wpp, wp, b[_PERM].reshape(1, O))
    z = z2.reshape(B, N, O)
    c = c2.reshape(B, N, O)
    idx_blk = neighbor_idx.reshape(NW, NCHUNK, CHUNK)
    out = _sc_gathermax(z, c, idx_blk)
    return out


# speedup vs baseline: 137.8117x; 1.0923x over previous
import functools

import jax
import jax.numpy as jnp
import numpy as np
from jax import lax
from jax.experimental import pallas as pl
from jax.experimental.pallas import tpu as pltpu
from jax.experimental.pallas import tpu_sc as plsc

B, N, K, C, O = 8, 4096, 32, 64, 64
LANES = 16
NW = 32
GB = 2
NG = B // GB
PPW = GB * N // NW
WPB = N // PPW

NBUF = 4
RPD = 128
PPD = RPD // K
NDESC = PPW // PPD
NTURN = NDESC // NBUF


def _mm_body(x_ref, pos_ref, wfp_ref, wpp_ref, wp_ref, b_ref, z_ref, c_ref):
    posb = pos_ref[0]
    zp = (jnp.dot(x_ref[0], wfp_ref[...],
                  preferred_element_type=jnp.float32)
          + jnp.dot(posb, wpp_ref[...],
                    preferred_element_type=jnp.float32)
          + b_ref[...])
    z_ref[0] = zp.astype(jnp.bfloat16)
    c_ref[0] = jnp.dot(posb, wp_ref[...],
                       preferred_element_type=jnp.float32)


def _make_tc_mm(g):
    return pl.pallas_call(
        _mm_body,
        grid=(GB,),
        in_specs=[
            pl.BlockSpec((1, N, C), lambda i: (g * GB + i, 0, 0)),
            pl.BlockSpec((1, N, 3), lambda i: (g * GB + i, 0, 0)),
            pl.BlockSpec((C, O), lambda i: (0, 0)),
            pl.BlockSpec((3, O), lambda i: (0, 0)),
            pl.BlockSpec((3, O), lambda i: (0, 0)),
            pl.BlockSpec((1, O), lambda i: (0, 0)),
        ],
        out_specs=[
            pl.BlockSpec((1, N, O), lambda i: (i, 0, 0)),
            pl.BlockSpec((1, N, O), lambda i: (i, 0, 0)),
        ],
        out_shape=[
            jax.ShapeDtypeStruct((GB, N, O), jnp.bfloat16),
            jax.ShapeDtypeStruct((GB, N, O), jnp.float32),
        ],
    )


_TC_MM = [_make_tc_mm(g) for g in range(NG)]


def _sc_body(z_hbm, c_hbm, idx_hbm, out_hbm, idx_all, rows_v, c_v, out_v,
             semg0, semg1, semg2, semg3, semo0, semo1, semo2, semo3):
    nc = 2
    wid = lax.axis_index("s") * nc + lax.axis_index("c")
    bb = wid // WPB
    lp_base = (wid % WPB) * PPW
    ztab = z_hbm.at[bb]
    ctab = c_hbm.at[bb]
    otab = out_hbm.at[bb]
    semg = (semg0, semg1, semg2, semg3)
    semo = (semo0, semo1, semo2, semo3)

    pltpu.sync_copy(idx_hbm.at[wid], idx_all)

    def fire(d, b):
        pltpu.async_copy(ztab.at[idx_all.at[d]], rows_v.at[b], semg[b])
        pltpu.async_copy(ctab.at[pl.ds(lp_base + d * PPD, PPD)], c_v.at[b],
                         semg[b])

    def drain(b):
        pltpu.make_async_copy(ztab.at[idx_all.at[0]], rows_v.at[b],
                              semg[b]).wait()
        pltpu.make_async_copy(ctab.at[pl.ds(0, PPD)], c_v.at[b],
                              semg[b]).wait()

    def fire_out(d, b):
        pltpu.async_copy(out_v.at[b], otab.at[pl.ds(lp_base + d * PPD, PPD)],
                         semo[b])

    def drain_out(b):
        pltpu.make_async_copy(out_v.at[b], otab.at[pl.ds(0, PPD)],
                              semo[b]).wait()

    def compute(b):
        for t in range(PPD):
            rr = t * K
            for g in range(O // 32):
                sl = pl.ds(g * 32, 32)
                acc = rows_v[b, rr, sl]
                for k in range(1, K):
                    acc = jnp.maximum(acc, rows_v[b, rr + k, sl])
                lo, hi = plsc.unpack(acc, format=plsc.PackFormat.INTERLEAVED)
                sl_lo = pl.ds(g * 32, LANES)
                sl_hi = pl.ds(g * 32 + LANES, LANES)
                out_v[b, t, sl_lo] = jnp.maximum(lo - c_v[b, t, sl_lo], 0.0)
                out_v[b, t, sl_hi] = jnp.maximum(hi - c_v[b, t, sl_hi], 0.0)

    for b in range(NBUF):
        fire(b, b)

    def turn(q, carry):
        for b in range(NBUF):
            d = NBUF * q + b
            drain(b)

            @pl.when(d >= NBUF)
            def _():
                drain_out(b)

            compute(b)
            fire_out(d, b)

            @pl.when(d + NBUF < NDESC)
            def _():
                fire(d + NBUF, b)
        return carry

    lax.fori_loop(0, NTURN, turn, 0)
    for b in range(NBUF):
        drain_out(b)


_sc_gathermax = functools.partial(
    pl.kernel,
    out_type=jax.ShapeDtypeStruct((GB, N, O), jnp.float32),
    mesh=plsc.VectorSubcoreMesh(core_axis_name="c", subcore_axis_name="s"),
    scratch_types=[
        pltpu.VMEM((NDESC, RPD), jnp.int32),
        pltpu.VMEM((NBUF, RPD, O), jnp.bfloat16),
        pltpu.VMEM((NBUF, PPD, O), jnp.float32),
        pltpu.VMEM((NBUF, PPD, O), jnp.float32),
        pltpu.SemaphoreType.DMA,
        pltpu.SemaphoreType.DMA,
        pltpu.SemaphoreType.DMA,
        pltpu.SemaphoreType.DMA,
        pltpu.SemaphoreType.DMA,
        pltpu.SemaphoreType.DMA,
        pltpu.SemaphoreType.DMA,
        pltpu.SemaphoreType.DMA,
    ],
    compiler_params=pltpu.CompilerParams(use_tc_tiling_on_sc=False,
                                         needs_layout_passes=False),
)(_sc_body)


_PERM = np.empty(O, dtype=np.int32)
for _g in range(O // 32):
    _PERM[_g * 32 + 0:_g * 32 + 32:2] = np.arange(16) + _g * 32
    _PERM[_g * 32 + 1:_g * 32 + 32:2] = np.arange(16) + _g * 32 + 16


def kernel(x, pos, neighbor_idx, W, b):
    wf = W[3:]
    wp = W[:3]
    wfp = wf[:, _PERM]
    wpp = wp[:, _PERM]
    bp = b[_PERM].reshape(1, O)
    idx_g = neighbor_idx.reshape(NG, NW, NDESC, RPD)
    outs = []
    for g in range(NG):
        z, c = _TC_MM[g](x, pos, wfp, wpp, wp, bp)
        outs.append(_sc_gathermax(z, c, idx_g[g]))
    return jnp.concatenate(outs, axis=0)
